# baseline (speedup 1.0000x reference)
import jax
import jax.numpy as jnp
from jax import lax
from jax.experimental import pallas as pl
from jax.experimental.pallas import tpu as pltpu
from jax.experimental.pallas import tpu_sc as plsc

N = 10000
D = 128
E = 320000
NC = 2
NS = 16
NSPLIT = N // NC
LOCAL_ROWS = 5120
TRASH = NSPLIT
CHUNK = 128
CHUNKS = 160
GROUPS = CHUNKS // 4
E_PAD = NS * CHUNKS * CHUNK
ROWS_PER_TILE = LOCAL_ROWS // NS
SLICE = 320
DEG_ROWS = NS * SLICE
TC_BLK = 1000


def _sc_body(x_hbm, src_hbm, dst_hbm, agg_out, deg_out,
             src_v, dst_v, rows_v, deg_local,
             agg_sh, sem_g0, sem_g1, sem_s0, sem_s1):
    c = lax.axis_index("c")
    s = lax.axis_index("s")
    base = s * ROWS_PER_TILE

    pltpu.sync_copy(src_hbm.at[s], src_v)
    pltpu.sync_copy(dst_hbm.at[c, s], dst_v)

    zer = jnp.zeros((16,), jnp.float32)
    one = jnp.ones((16,), jnp.float32)

    def fill_rows(r, carry):
        for j0 in range(D // 16):
            rows_v[0, r, pl.ds(j0 * 16, 16)] = zer
        return carry

    lax.fori_loop(0, CHUNK, fill_rows, 0)

    def fill_deg(r, carry):
        deg_local[pl.ds(r * 16, 16)] = zer
        return carry

    lax.fori_loop(0, DEG_ROWS // 16, fill_deg, 0)

    pltpu.sync_copy(rows_v.at[0], agg_sh.at[pl.ds(base, CHUNK)])
    pltpu.sync_copy(rows_v.at[0], agg_sh.at[pl.ds(base + CHUNK, CHUNK)])
    pltpu.sync_copy(rows_v.at[0, pl.ds(0, 64)],
                    agg_sh.at[pl.ds(base + 2 * CHUNK, 64)])
    plsc.subcore_barrier()

    def hist(j):
        for k in range(CHUNK // 16):
            idx = dst_v[j, pl.ds(k * 16, 16)]
            plsc.addupdate_scatter(deg_local, [idx], one)

    def gath(j, buf, sem):
        pltpu.async_copy(x_hbm.at[src_v.at[j]], rows_v.at[buf], sem)

    def gath_wait(j, buf, sem):
        pltpu.make_async_copy(x_hbm.at[src_v.at[j]], rows_v.at[buf],
                              sem).wait()

    def scat(j, buf, sem):
        pltpu.async_copy(rows_v.at[buf], agg_sh.at[dst_v.at[j]], sem,
                         add=True)

    def scat_wait(j, buf, sem):
        pltpu.make_async_copy(rows_v.at[buf], agg_sh.at[dst_v.at[j]],
                              sem).wait()

    gath(0, 0, sem_g0)
    gath(1, 1, sem_g1)

    def body(i, carry):
        j0 = 2 * i

        def half(j, buf, sem_g, sem_s):
            gath_wait(j, buf, sem_g)
            scat(j, buf, sem_s)
            hist(j)
            scat_wait(j, buf, sem_s)

            @pl.when(j + 2 < CHUNKS)
            def _():
                gath(j + 2, buf, sem_g)

        half(j0, 0, sem_g0, sem_s0)
        half(j0 + 1, 1, sem_g1, sem_s1)
        return carry

    lax.fori_loop(0, CHUNKS // 2, body, 0)

    plsc.subcore_barrier()
    pltpu.sync_copy(deg_local,
                    deg_out.at[pl.ds((c * NS + s) * DEG_ROWS, DEG_ROWS)])
    pltpu.sync_copy(agg_sh.at[pl.ds(base, ROWS_PER_TILE)],
                    agg_out.at[c, pl.ds(base, ROWS_PER_TILE)])


@jax.jit
def _sc_scatter(x, src3, dst4):
    mesh = plsc.VectorSubcoreMesh(core_axis_name="c", subcore_axis_name="s")
    return pl.kernel(
        _sc_body,
        mesh=mesh,
        compiler_params=pltpu.CompilerParams(needs_layout_passes=False),
        out_type=[
            jax.ShapeDtypeStruct((NC, LOCAL_ROWS, D), jnp.float32),
            jax.ShapeDtypeStruct((NC * NS * DEG_ROWS,), jnp.float32),
        ],
        scratch_types=[
            pltpu.VMEM((CHUNKS, CHUNK), jnp.int32),
            pltpu.VMEM((CHUNKS, CHUNK), jnp.int32),
            pltpu.VMEM((2, CHUNK, D), jnp.float32),
            pltpu.VMEM((DEG_ROWS,), jnp.float32),
            pltpu.VMEM_SHARED((LOCAL_ROWS, D), jnp.float32),
            pltpu.SemaphoreType.DMA,
            pltpu.SemaphoreType.DMA,
            pltpu.SemaphoreType.DMA,
            pltpu.SemaphoreType.DMA,
        ],
    )(x, src3, dst4)


def _tc_body(agg_ref, deg_ref, W_ref, b_ref, out_ref):
    deg = jnp.sum(deg_ref[0, 0], axis=0)[:, None]
    h = agg_ref[0] / jnp.maximum(deg, 1.0)
    out_ref[...] = (
        jnp.dot(h, W_ref[...], preferred_element_type=jnp.float32) + b_ref[...]
    )


@jax.jit
def _tc_finish(agg2, deg_full, W, b2):
    return pl.pallas_call(
        _tc_body,
        grid=(N // TC_BLK,),
        in_specs=[
            pl.BlockSpec((1, TC_BLK, D), lambda i: (i // 5, i % 5, 0)),
            pl.BlockSpec((1, 1, NS, TC_BLK), lambda i: (i // 5, i % 5, 0, 0)),
            pl.BlockSpec((D, D), lambda i: (0, 0)),
            pl.BlockSpec((1, D), lambda i: (0, 0)),
        ],
        out_specs=pl.BlockSpec((TC_BLK, D), lambda i: (i, 0)),
        out_shape=jax.ShapeDtypeStruct((N, D), jnp.float32),
    )(agg2, deg_full, W, b2)


def kernel(x, edge_index, p_map, W, b):
    del p_map
    src = edge_index[0].astype(jnp.int32)
    dst = edge_index[1].astype(jnp.int32)
    pad = E_PAD - E
    src3 = jnp.concatenate([src, jnp.zeros((pad,), jnp.int32)]).reshape(
        NS, CHUNKS, CHUNK)
    dst_p = jnp.concatenate([dst, jnp.full((pad,), -1, jnp.int32)])
    locs = []
    for core in range(NC):
        local = dst_p - core * NSPLIT
        ok = (local >= 0) & (local < NSPLIT)
        locs.append(jnp.where(ok, local, TRASH))
    dst4 = jnp.stack(locs).reshape(NC, NS, CHUNKS, CHUNK)
    agg2, deg2 = _sc_scatter(x, src3, dst4)
    deg4 = deg2.reshape(NC, NS, DEG_ROWS)[:, :, :NSPLIT].reshape(
        NC, NS, N // (NC * TC_BLK), TC_BLK).transpose(0, 2, 1, 3)
    return _tc_finish(agg2, deg4, W, b.reshape(1, D))


# speedup vs baseline: 2.0928x; 2.0928x over previous
import jax
import jax.numpy as jnp
from jax import lax
from jax.experimental import pallas as pl
from jax.experimental.pallas import tpu as pltpu
from jax.experimental.pallas import tpu_sc as plsc

N = 10000
D = 128
E = 320000
NC = 2
NS = 16
NSPLIT = N // NC
LOCAL_ROWS = 5120
TRASH = NSPLIT
CHUNK = 128
CHUNKS = 157
E_PAD = NS * CHUNKS * CHUNK
ROWS_PER_TILE = LOCAL_ROWS // NS
SLICE = 320
DEG_ROWS = NS * SLICE
TC_BLK = 1000


def _sc_body(x_hbm, src_hbm, dst_hbm, agg_out, deg_out,
             src_v, dst_v, rows_v, deg_local,
             agg_sh, sem_g0, sem_g1, sem_s0, sem_s1):
    c = lax.axis_index("c")
    s = lax.axis_index("s")
    base = s * ROWS_PER_TILE

    pltpu.sync_copy(src_hbm.at[s], src_v)
    pltpu.sync_copy(dst_hbm.at[c, s], dst_v)

    zer = jnp.zeros((16,), jnp.float32)
    one = jnp.ones((16,), jnp.float32)

    def fill_rows(r, carry):
        for j0 in range(D // 16):
            rows_v[0, r, pl.ds(j0 * 16, 16)] = zer
        return carry

    lax.fori_loop(0, CHUNK, fill_rows, 0)

    def fill_deg(r, carry):
        deg_local[pl.ds(r * 16, 16)] = zer
        return carry

    lax.fori_loop(0, DEG_ROWS // 16, fill_deg, 0)

    pltpu.sync_copy(rows_v.at[0], agg_sh.at[pl.ds(base, CHUNK)])
    pltpu.sync_copy(rows_v.at[0], agg_sh.at[pl.ds(base + CHUNK, CHUNK)])
    pltpu.sync_copy(rows_v.at[0, pl.ds(0, 64)],
                    agg_sh.at[pl.ds(base + 2 * CHUNK, 64)])
    plsc.subcore_barrier()

    def hist(j):
        for k in range(CHUNK // 16):
            idx = dst_v[j, pl.ds(k * 16, 16)]
            plsc.addupdate_scatter(deg_local, [idx], one)

    def gath(j, buf, sem):
        pltpu.async_copy(x_hbm.at[src_v.at[j]], rows_v.at[buf], sem)

    def gath_wait(j, buf, sem):
        pltpu.make_async_copy(x_hbm.at[src_v.at[j]], rows_v.at[buf],
                              sem).wait()

    def scat(j, buf, sem):
        pltpu.async_copy(rows_v.at[buf], agg_sh.at[dst_v.at[j]], sem,
                         add=True)

    def scat_wait(j, buf, sem):
        pltpu.make_async_copy(rows_v.at[buf], agg_sh.at[dst_v.at[j]],
                              sem).wait()

    gath(0, 0, sem_g0)

    def body(i, carry):
        j0 = 2 * i
        j1 = j0 + 1
        j2 = j0 + 2
        gath(j1, 1, sem_g1)
        gath_wait(j0, 0, sem_g0)
        pltpu.sync_copy(rows_v.at[0], agg_sh.at[dst_v.at[j0]], add=True)
        gath(j2, 0, sem_g0)
        hist(j0)
        gath_wait(j1, 1, sem_g1)
        pltpu.sync_copy(rows_v.at[1], agg_sh.at[dst_v.at[j1]], add=True)
        hist(j1)
        return carry

    lax.fori_loop(0, (CHUNKS - 1) // 2, body, 0)

    j_last = CHUNKS - 1
    gath_wait(j_last, 0, sem_g0)
    pltpu.sync_copy(rows_v.at[0], agg_sh.at[dst_v.at[j_last]], add=True)
    hist(j_last)

    plsc.subcore_barrier()
    pltpu.sync_copy(deg_local,
                    deg_out.at[pl.ds((c * NS + s) * DEG_ROWS, DEG_ROWS)])
    pltpu.sync_copy(agg_sh.at[pl.ds(base, ROWS_PER_TILE)],
                    agg_out.at[c, pl.ds(base, ROWS_PER_TILE)])


@jax.jit
def _sc_scatter(x, src3, dst4):
    mesh = plsc.VectorSubcoreMesh(core_axis_name="c", subcore_axis_name="s")
    return pl.kernel(
        _sc_body,
        mesh=mesh,
        compiler_params=pltpu.CompilerParams(needs_layout_passes=False),
        out_type=[
            jax.ShapeDtypeStruct((NC, LOCAL_ROWS, D), jnp.float32),
            jax.ShapeDtypeStruct((NC * NS * DEG_ROWS,), jnp.float32),
        ],
        scratch_types=[
            pltpu.VMEM((CHUNKS, CHUNK), jnp.int32),
            pltpu.VMEM((CHUNKS, CHUNK), jnp.int32),
            pltpu.VMEM((2, CHUNK, D), jnp.float32),
            pltpu.VMEM((DEG_ROWS,), jnp.float32),
            pltpu.VMEM_SHARED((LOCAL_ROWS, D), jnp.float32),
            pltpu.SemaphoreType.DMA,
            pltpu.SemaphoreType.DMA,
            pltpu.SemaphoreType.DMA,
            pltpu.SemaphoreType.DMA,
        ],
    )(x, src3, dst4)


def _tc_body(agg_ref, deg_ref, W_ref, b_ref, out_ref):
    deg = jnp.sum(deg_ref[0, 0], axis=0)[:, None]
    h = agg_ref[0] / jnp.maximum(deg, 1.0)
    out_ref[...] = (
        jnp.dot(h, W_ref[...], preferred_element_type=jnp.float32) + b_ref[...]
    )


@jax.jit
def _tc_finish(agg2, deg_full, W, b2):
    return pl.pallas_call(
        _tc_body,
        grid=(N // TC_BLK,),
        in_specs=[
            pl.BlockSpec((1, TC_BLK, D), lambda i: (i // 5, i % 5, 0)),
            pl.BlockSpec((1, 1, NS, TC_BLK), lambda i: (i // 5, i % 5, 0, 0)),
            pl.BlockSpec((D, D), lambda i: (0, 0)),
            pl.BlockSpec((1, D), lambda i: (0, 0)),
        ],
        out_specs=pl.BlockSpec((TC_BLK, D), lambda i: (i, 0)),
        out_shape=jax.ShapeDtypeStruct((N, D), jnp.float32),
    )(agg2, deg_full, W, b2)


def kernel(x, edge_index, p_map, W, b):
    del p_map
    src = edge_index[0].astype(jnp.int32)
    dst = edge_index[1].astype(jnp.int32)
    pad = E_PAD - E
    src3 = jnp.concatenate([src, jnp.zeros((pad,), jnp.int32)]).reshape(
        NS, CHUNKS, CHUNK)
    dst_p = jnp.concatenate([dst, jnp.full((pad,), -1, jnp.int32)])
    locs = []
    for core in range(NC):
        local = dst_p - core * NSPLIT
        ok = (local >= 0) & (local < NSPLIT)
        locs.append(jnp.where(ok, local, TRASH))
    dst4 = jnp.stack(locs).reshape(NC, NS, CHUNKS, CHUNK)
    agg2, deg2 = _sc_scatter(x, src3, dst4)
    deg4 = deg2.reshape(NC, NS, DEG_ROWS)[:, :, :NSPLIT].reshape(
        NC, NS, N // (NC * TC_BLK), TC_BLK).transpose(0, 2, 1, 3)
    return _tc_finish(agg2, deg4, W, b.reshape(1, D))


# speedup vs baseline: 3.6180x; 1.7288x over previous
import jax
import jax.numpy as jnp
from jax import lax
from jax.experimental import pallas as pl
from jax.experimental.pallas import tpu as pltpu
from jax.experimental.pallas import tpu_sc as plsc

N = 10000
D = 128
E = 320000
NC = 2
NS = 16
NSPLIT = N // NC
LOCAL_ROWS = 5120
TRASH = NSPLIT
CHUNK = 128
CHUNKS = 157
CAP = 160
E_PAD = NS * CHUNKS * CHUNK
ROWS_PER_TILE = LOCAL_ROWS // NS
SLICE = 320
DEG_ROWS = NS * SLICE
TC_BLK = 1000


def _sc_body(x_hbm, src_hbm, dst_hbm, agg_out, deg_out,
             src_c, dst_c, rows_v, deg_local,
             agg_sh, sem_g0, sem_g1, sem_s0, sem_s1):
    c = lax.axis_index("c")
    s = lax.axis_index("s")
    base = s * ROWS_PER_TILE

    nwords = CHUNKS * CHUNK
    pltpu.sync_copy(src_hbm.at[pl.ds(pl.multiple_of(s * nwords, 128), nwords)],
                    src_c.at[pl.ds(0, nwords)])
    pltpu.sync_copy(
        dst_hbm.at[pl.ds(pl.multiple_of((c * NS + s) * nwords, 128), nwords)],
        dst_c.at[pl.ds(0, nwords)])

    zer = jnp.zeros((16,), jnp.float32)
    one = jnp.ones((16,), jnp.float32)

    def fill_rows(r, carry):
        for j0 in range(D // 16):
            rows_v[0, r, pl.ds(j0 * 16, 16)] = zer
        return carry

    lax.fori_loop(0, CHUNK, fill_rows, 0)

    def fill_deg(r, carry):
        deg_local[pl.ds(r * 16, 16)] = zer
        return carry

    lax.fori_loop(0, DEG_ROWS // 16, fill_deg, 0)

    pltpu.sync_copy(rows_v.at[0], agg_sh.at[pl.ds(base, CHUNK)])
    pltpu.sync_copy(rows_v.at[0], agg_sh.at[pl.ds(base + CHUNK, CHUNK)])
    pltpu.sync_copy(rows_v.at[0, pl.ds(0, 64)],
                    agg_sh.at[pl.ds(base + 2 * CHUNK, 64)])
    plsc.subcore_barrier()

    def comp(r, cur):
        dvec = dst_c[pl.ds(r * 16, 16)]
        svec = src_c[pl.ds(r * 16, 16)]
        m = dvec < NSPLIT
        plsc.store_compressed(dst_c.at[pl.ds(cur, 16)], dvec, mask=m)
        plsc.store_compressed(src_c.at[pl.ds(cur, 16)], svec, mask=m)
        return cur + jnp.sum(m.astype(jnp.int32))

    cnt = lax.fori_loop(0, CHUNKS * CHUNK // 16, comp, 0)

    cur_a = cnt & ~15
    rem = cnt - cur_a
    ii = lax.iota(jnp.int32, 16)
    trash_v = jnp.full((16,), TRASH, jnp.int32)
    zer_i = jnp.zeros((16,), jnp.int32)
    dst_c[pl.ds(cur_a, 16)] = jnp.where(ii >= rem, TRASH,
                                        dst_c[pl.ds(cur_a, 16)])
    src_c[pl.ds(cur_a, 16)] = jnp.where(ii >= rem, 0,
                                        src_c[pl.ds(cur_a, 16)])
    for k in range(1, 17):
        dst_c[pl.ds(cur_a + k * 16, 16)] = trash_v
        src_c[pl.ds(cur_a + k * 16, 16)] = zer_i

    n_chunks = jnp.maximum((cnt + CHUNK - 1) // CHUNK, 1)

    def hist(j):
        for k in range(CHUNK // 16):
            idx = dst_c[pl.ds(j * CHUNK + k * 16, 16)]
            plsc.addupdate_scatter(deg_local, [idx], one)

    def gath(j, buf, sem):
        pltpu.async_copy(x_hbm.at[src_c.at[pl.ds(j * CHUNK, CHUNK)]],
                         rows_v.at[buf], sem)

    def gath_wait(j, buf, sem):
        pltpu.make_async_copy(x_hbm.at[src_c.at[pl.ds(j * CHUNK, CHUNK)]],
                              rows_v.at[buf], sem).wait()

    def scat(j, buf):
        pltpu.sync_copy(rows_v.at[buf],
                        agg_sh.at[dst_c.at[pl.ds(j * CHUNK, CHUNK)]],
                        add=True)

    gath(0, 0, sem_g0)

    def body(i, carry):
        j0 = 2 * i
        j1 = j0 + 1
        j2 = j0 + 2

        @pl.when(j1 < n_chunks)
        def _():
            gath(j1, 1, sem_g1)

        gath_wait(j0, 0, sem_g0)
        scat(j0, 0)

        @pl.when(j2 < n_chunks)
        def _():
            gath(j2, 0, sem_g0)

        hist(j0)

        @pl.when(j1 < n_chunks)
        def _():
            gath_wait(j1, 1, sem_g1)
            scat(j1, 1)
            hist(j1)

        return carry

    lax.fori_loop(0, (n_chunks + 1) // 2, body, 0)

    plsc.subcore_barrier()
    pltpu.sync_copy(deg_local,
                    deg_out.at[pl.ds((c * NS + s) * DEG_ROWS, DEG_ROWS)])
    pltpu.sync_copy(agg_sh.at[pl.ds(base, ROWS_PER_TILE)],
                    agg_out.at[c, pl.ds(base, ROWS_PER_TILE)])


@jax.jit
def _sc_scatter(x, src3, dst4):
    mesh = plsc.VectorSubcoreMesh(core_axis_name="c", subcore_axis_name="s")
    return pl.kernel(
        _sc_body,
        mesh=mesh,
        compiler_params=pltpu.CompilerParams(needs_layout_passes=False),
        out_type=[
            jax.ShapeDtypeStruct((NC, LOCAL_ROWS, D), jnp.float32),
            jax.ShapeDtypeStruct((NC * NS * DEG_ROWS,), jnp.float32),
        ],
        scratch_types=[
            pltpu.VMEM((CAP * CHUNK,), jnp.int32),
            pltpu.VMEM((CAP * CHUNK,), jnp.int32),
            pltpu.VMEM((2, CHUNK, D), jnp.float32),
            pltpu.VMEM((DEG_ROWS,), jnp.float32),
            pltpu.VMEM_SHARED((LOCAL_ROWS, D), jnp.float32),
            pltpu.SemaphoreType.DMA,
            pltpu.SemaphoreType.DMA,
            pltpu.SemaphoreType.DMA,
            pltpu.SemaphoreType.DMA,
        ],
    )(x, src3, dst4)


def _tc_body(agg_ref, deg_ref, W_ref, b_ref, out_ref):
    deg = jnp.sum(deg_ref[0, 0], axis=0)[:, None]
    h = agg_ref[0] / jnp.maximum(deg, 1.0)
    out_ref[...] = (
        jnp.dot(h, W_ref[...], preferred_element_type=jnp.float32) + b_ref[...]
    )


@jax.jit
def _tc_finish(agg2, deg_full, W, b2):
    return pl.pallas_call(
        _tc_body,
        grid=(N // TC_BLK,),
        in_specs=[
            pl.BlockSpec((1, TC_BLK, D), lambda i: (i // 5, i % 5, 0)),
            pl.BlockSpec((1, 1, NS, TC_BLK), lambda i: (i // 5, i % 5, 0, 0)),
            pl.BlockSpec((D, D), lambda i: (0, 0)),
            pl.BlockSpec((1, D), lambda i: (0, 0)),
        ],
        out_specs=pl.BlockSpec((TC_BLK, D), lambda i: (i, 0)),
        out_shape=jax.ShapeDtypeStruct((N, D), jnp.float32),
    )(agg2, deg_full, W, b2)


def kernel(x, edge_index, p_map, W, b):
    del p_map
    src = edge_index[0].astype(jnp.int32)
    dst = edge_index[1].astype(jnp.int32)
    pad = E_PAD - E
    src3 = jnp.concatenate([src, jnp.zeros((pad,), jnp.int32)])
    dst_p = jnp.concatenate([dst, jnp.full((pad,), -1, jnp.int32)])
    locs = []
    for core in range(NC):
        local = dst_p - core * NSPLIT
        ok = (local >= 0) & (local < NSPLIT)
        locs.append(jnp.where(ok, local, TRASH))
    dst4 = jnp.stack(locs).reshape(-1)
    agg2, deg2 = _sc_scatter(x, src3, dst4)
    deg4 = deg2.reshape(NC, NS, DEG_ROWS)[:, :, :NSPLIT].reshape(
        NC, NS, N // (NC * TC_BLK), TC_BLK).transpose(0, 2, 1, 3)
    return _tc_finish(agg2, deg4, W, b.reshape(1, D))


# speedup vs baseline: 3.6493x; 1.0087x over previous
import jax
import jax.numpy as jnp
from jax import lax
from jax.experimental import pallas as pl
from jax.experimental.pallas import tpu as pltpu
from jax.experimental.pallas import tpu_sc as plsc

N = 10000
D = 128
E = 320000
NC = 2
NS = 16
NSPLIT = N // NC
LOCAL_ROWS = 5120
TRASH = NSPLIT
CHUNK = 128
CHUNKS = 157
CAP = 160
E_PAD = NS * CHUNKS * CHUNK
ROWS_PER_TILE = LOCAL_ROWS // NS
SLICE = 320
DEG_ROWS = NS * SLICE
TC_BLK = 1000


def _sc_body(x_hbm, src_hbm, dst_hbm, agg_out, deg_out,
             src_c, dst_c, rows_v, deg_local,
             agg_sh, sem_g0, sem_g1, sem_s0, sem_s1):
    c = lax.axis_index("c")
    s = lax.axis_index("s")
    base = s * ROWS_PER_TILE

    nwords = CHUNKS * CHUNK
    pltpu.sync_copy(src_hbm.at[pl.ds(pl.multiple_of(s * nwords, 128), nwords)],
                    src_c.at[pl.ds(0, nwords)])
    pltpu.sync_copy(dst_hbm.at[pl.ds(pl.multiple_of(s * nwords, 128), nwords)],
                    dst_c.at[pl.ds(0, nwords)])

    zer = jnp.zeros((16,), jnp.float32)
    one = jnp.ones((16,), jnp.float32)

    def fill_rows(r, carry):
        for j0 in range(D // 16):
            rows_v[0, r, pl.ds(j0 * 16, 16)] = zer
        return carry

    lax.fori_loop(0, CHUNK, fill_rows, 0)

    def fill_deg(r, carry):
        deg_local[pl.ds(r * 16, 16)] = zer
        return carry

    lax.fori_loop(0, DEG_ROWS // 16, fill_deg, 0)

    pltpu.sync_copy(rows_v.at[0], agg_sh.at[pl.ds(base, CHUNK)])
    pltpu.sync_copy(rows_v.at[0], agg_sh.at[pl.ds(base + CHUNK, CHUNK)])
    pltpu.sync_copy(rows_v.at[0, pl.ds(0, 64)],
                    agg_sh.at[pl.ds(base + 2 * CHUNK, 64)])
    plsc.subcore_barrier()

    cbase = c * NSPLIT

    def comp(r, cur):
        dvec = dst_c[pl.ds(r * 16, 16)] - cbase
        svec = src_c[pl.ds(r * 16, 16)]
        m = (dvec >= 0) & (dvec < NSPLIT)
        plsc.store_compressed(dst_c.at[pl.ds(cur, 16)], dvec, mask=m)
        plsc.store_compressed(src_c.at[pl.ds(cur, 16)], svec, mask=m)
        return cur + jnp.sum(m.astype(jnp.int32))

    cnt = lax.fori_loop(0, CHUNKS * CHUNK // 16, comp, 0)

    cur_a = cnt & ~15
    rem = cnt - cur_a
    ii = lax.iota(jnp.int32, 16)
    trash_v = jnp.full((16,), TRASH, jnp.int32)
    zer_i = jnp.zeros((16,), jnp.int32)
    dst_c[pl.ds(cur_a, 16)] = jnp.where(ii >= rem, TRASH,
                                        dst_c[pl.ds(cur_a, 16)])
    src_c[pl.ds(cur_a, 16)] = jnp.where(ii >= rem, 0,
                                        src_c[pl.ds(cur_a, 16)])
    for k in range(1, 17):
        dst_c[pl.ds(cur_a + k * 16, 16)] = trash_v
        src_c[pl.ds(cur_a + k * 16, 16)] = zer_i

    n_chunks = jnp.maximum((cnt + CHUNK - 1) // CHUNK, 1)

    def hist(j):
        for k in range(CHUNK // 16):
            idx = dst_c[pl.ds(j * CHUNK + k * 16, 16)]
            plsc.addupdate_scatter(deg_local, [idx], one)

    def gath(j, buf, sem):
        pltpu.async_copy(x_hbm.at[src_c.at[pl.ds(j * CHUNK, CHUNK)]],
                         rows_v.at[buf], sem)

    def gath_wait(j, buf, sem):
        pltpu.make_async_copy(x_hbm.at[src_c.at[pl.ds(j * CHUNK, CHUNK)]],
                              rows_v.at[buf], sem).wait()

    def scat(j, buf):
        pltpu.sync_copy(rows_v.at[buf],
                        agg_sh.at[dst_c.at[pl.ds(j * CHUNK, CHUNK)]],
                        add=True)

    gath(0, 0, sem_g0)

    def body(i, carry):
        j0 = 2 * i
        j1 = j0 + 1
        j2 = j0 + 2

        @pl.when(j1 < n_chunks)
        def _():
            gath(j1, 1, sem_g1)

        gath_wait(j0, 0, sem_g0)
        scat(j0, 0)

        @pl.when(j2 < n_chunks)
        def _():
            gath(j2, 0, sem_g0)

        hist(j0)

        @pl.when(j1 < n_chunks)
        def _():
            gath_wait(j1, 1, sem_g1)
            scat(j1, 1)
            hist(j1)

        return carry

    lax.fori_loop(0, (n_chunks + 1) // 2, body, 0)

    plsc.subcore_barrier()
    pltpu.sync_copy(deg_local,
                    deg_out.at[pl.ds((c * NS + s) * DEG_ROWS, DEG_ROWS)])
    pltpu.sync_copy(agg_sh.at[pl.ds(base, ROWS_PER_TILE)],
                    agg_out.at[c, pl.ds(base, ROWS_PER_TILE)])


@jax.jit
def _sc_scatter(x, src3, dst4):
    mesh = plsc.VectorSubcoreMesh(core_axis_name="c", subcore_axis_name="s")
    return pl.kernel(
        _sc_body,
        mesh=mesh,
        compiler_params=pltpu.CompilerParams(needs_layout_passes=False),
        out_type=[
            jax.ShapeDtypeStruct((NC, LOCAL_ROWS, D), jnp.float32),
            jax.ShapeDtypeStruct((NC * NS * DEG_ROWS,), jnp.float32),
        ],
        scratch_types=[
            pltpu.VMEM((CAP * CHUNK,), jnp.int32),
            pltpu.VMEM((CAP * CHUNK,), jnp.int32),
            pltpu.VMEM((2, CHUNK, D), jnp.float32),
            pltpu.VMEM((DEG_ROWS,), jnp.float32),
            pltpu.VMEM_SHARED((LOCAL_ROWS, D), jnp.float32),
            pltpu.SemaphoreType.DMA,
            pltpu.SemaphoreType.DMA,
            pltpu.SemaphoreType.DMA,
            pltpu.SemaphoreType.DMA,
        ],
    )(x, src3, dst4)


def _tc_body(agg_ref, deg_ref, W_ref, b_ref, out_ref):
    deg = jnp.sum(deg_ref[0, 0], axis=0)[:, None]
    h = agg_ref[0] / jnp.maximum(deg, 1.0)
    out_ref[...] = (
        jnp.dot(h, W_ref[...], preferred_element_type=jnp.float32) + b_ref[...]
    )


@jax.jit
def _tc_finish(agg2, deg_full, W, b2):
    return pl.pallas_call(
        _tc_body,
        grid=(N // TC_BLK,),
        in_specs=[
            pl.BlockSpec((1, TC_BLK, D), lambda i: (i // 5, i % 5, 0)),
            pl.BlockSpec((1, 1, NS, TC_BLK), lambda i: (i // 5, i % 5, 0, 0)),
            pl.BlockSpec((D, D), lambda i: (0, 0)),
            pl.BlockSpec((1, D), lambda i: (0, 0)),
        ],
        out_specs=pl.BlockSpec((TC_BLK, D), lambda i: (i, 0)),
        out_shape=jax.ShapeDtypeStruct((N, D), jnp.float32),
    )(agg2, deg_full, W, b2)


def kernel(x, edge_index, p_map, W, b):
    del p_map
    src = edge_index[0].astype(jnp.int32)
    dst = edge_index[1].astype(jnp.int32)
    pad = E_PAD - E
    src3 = jnp.concatenate([src, jnp.zeros((pad,), jnp.int32)])
    dst4 = jnp.concatenate([dst, jnp.full((pad,), -1, jnp.int32)])
    agg2, deg2 = _sc_scatter(x, src3, dst4)
    deg4 = deg2.reshape(NC, NS, DEG_ROWS)[:, :, :NSPLIT].reshape(
        NC, NS, N // (NC * TC_BLK), TC_BLK).transpose(0, 2, 1, 3)
    return _tc_finish(agg2, deg4, W, b.reshape(1, D))


# speedup vs baseline: 3.8165x; 1.0458x over previous
import jax
import jax.numpy as jnp
from jax import lax
from jax.experimental import pallas as pl
from jax.experimental.pallas import tpu as pltpu
from jax.experimental.pallas import tpu_sc as plsc

N = 10000
D = 128
E = 320000
NC = 2
NS = 16
NSPLIT = N // NC
LOCAL_ROWS = 5120
TRASH = NSPLIT
CHUNK = 128
CHUNKS = 157
CAP = 160
E_PAD = NS * CHUNKS * CHUNK
ROWS_PER_TILE = LOCAL_ROWS // NS
SLICE = 320
DEG_ROWS = NS * SLICE
TC_BLK = 1000


def _sc_body(x_hbm, pk_hbm, agg_out, deg_out,
             pk_c, src_idx, dst_idx, rows_v, deg_local,
             agg_sh, sem_g0, sem_g1, sem_g2, sem_s0):
    c = lax.axis_index("c")
    s = lax.axis_index("s")
    base = s * ROWS_PER_TILE

    nwords = CHUNKS * CHUNK
    pltpu.sync_copy(pk_hbm.at[pl.ds(pl.multiple_of(s * nwords, 128), nwords)],
                    pk_c.at[pl.ds(0, nwords)])

    zer = jnp.zeros((16,), jnp.float32)
    one = jnp.ones((16,), jnp.float32)

    def fill_rows(r, carry):
        for j0 in range(D // 16):
            rows_v[0, r, pl.ds(j0 * 16, 16)] = zer
        return carry

    lax.fori_loop(0, CHUNK, fill_rows, 0)

    def fill_deg(r, carry):
        deg_local[pl.ds(r * 16, 16)] = zer
        return carry

    lax.fori_loop(0, DEG_ROWS // 16, fill_deg, 0)

    pltpu.sync_copy(rows_v.at[0], agg_sh.at[pl.ds(base, CHUNK)])
    pltpu.sync_copy(rows_v.at[0], agg_sh.at[pl.ds(base + CHUNK, CHUNK)])
    pltpu.sync_copy(rows_v.at[0, pl.ds(0, 64)],
                    agg_sh.at[pl.ds(base + 2 * CHUNK, 64)])
    plsc.subcore_barrier()

    cbase = c * NSPLIT

    def comp(r, cur):
        pvec = pk_c[pl.ds(r * 16, 16)]
        svec = pvec & 16383
        dloc = (pvec >> 14) - (1 + cbase)
        m = (dloc >= 0) & (dloc < NSPLIT)
        plsc.store_compressed(pk_c.at[pl.ds(cur, 16)],
                              svec | (dloc << 14), mask=m)
        return cur + jnp.sum(m.astype(jnp.int32))

    cnt = lax.fori_loop(0, CHUNKS * CHUNK // 16, comp, 0)

    cur_a = cnt & ~15
    rem = cnt - cur_a
    ii = lax.iota(jnp.int32, 16)
    trash_v = jnp.full((16,), TRASH << 14, jnp.int32)
    pk_c[pl.ds(cur_a, 16)] = jnp.where(ii >= rem, TRASH << 14,
                                       pk_c[pl.ds(cur_a, 16)])
    for k in range(1, 17):
        pk_c[pl.ds(cur_a + k * 16, 16)] = trash_v

    n_chunks = jnp.maximum((cnt + CHUNK - 1) // CHUNK, 1)

    def hist(j):
        del j
        for k in range(CHUNK // 16):
            idx = dst_idx[pl.ds(k * 16, 16)]
            plsc.addupdate_scatter(deg_local, [idx], one)

    def gath(j, buf, sem):
        for k in range(CHUNK // 16):
            src_idx[buf, pl.ds(k * 16, 16)] = (
                pk_c[pl.ds(j * CHUNK + k * 16, 16)] & 16383)
        pltpu.async_copy(x_hbm.at[src_idx.at[buf]], rows_v.at[buf], sem)

    def gath_wait(j, buf, sem):
        del j
        pltpu.make_async_copy(x_hbm.at[src_idx.at[buf]], rows_v.at[buf],
                              sem).wait()

    def scat(j, buf):
        for k in range(CHUNK // 16):
            dst_idx[pl.ds(k * 16, 16)] = (
                pk_c[pl.ds(j * CHUNK + k * 16, 16)] >> 14)
        pltpu.sync_copy(rows_v.at[buf], agg_sh.at[dst_idx], add=True)

    gath(0, 0, sem_g0)

    @pl.when(1 < n_chunks)
    def _():
        gath(1, 1, sem_g1)

    def body(i, carry):
        j0 = 3 * i
        j1 = j0 + 1
        j2 = j0 + 2
        j3 = j0 + 3
        j4 = j0 + 4

        @pl.when(j2 < n_chunks)
        def _():
            gath(j2, 2, sem_g2)

        gath_wait(j0, 0, sem_g0)
        scat(j0, 0)

        @pl.when(j3 < n_chunks)
        def _():
            gath(j3, 0, sem_g0)

        hist(j0)

        @pl.when(j1 < n_chunks)
        def _():
            gath_wait(j1, 1, sem_g1)
            scat(j1, 1)

            @pl.when(j4 < n_chunks)
            def _():
                gath(j4, 1, sem_g1)

            hist(j1)

        @pl.when(j2 < n_chunks)
        def _():
            gath_wait(j2, 2, sem_g2)
            scat(j2, 2)
            hist(j2)

        return carry

    lax.fori_loop(0, (n_chunks + 2) // 3, body, 0)

    plsc.subcore_barrier()
    pltpu.sync_copy(deg_local,
                    deg_out.at[pl.ds((c * NS + s) * DEG_ROWS, DEG_ROWS)])
    pltpu.sync_copy(agg_sh.at[pl.ds(base, ROWS_PER_TILE)],
                    agg_out.at[c, pl.ds(base, ROWS_PER_TILE)])


@jax.jit
def _sc_scatter(x, packed):
    mesh = plsc.VectorSubcoreMesh(core_axis_name="c", subcore_axis_name="s")
    return pl.kernel(
        _sc_body,
        mesh=mesh,
        compiler_params=pltpu.CompilerParams(needs_layout_passes=False),
        out_type=[
            jax.ShapeDtypeStruct((NC, LOCAL_ROWS, D), jnp.float32),
            jax.ShapeDtypeStruct((NC * NS * DEG_ROWS,), jnp.float32),
        ],
        scratch_types=[
            pltpu.VMEM((CAP * CHUNK,), jnp.int32),
            pltpu.VMEM((3, CHUNK), jnp.int32),
            pltpu.VMEM((CHUNK,), jnp.int32),
            pltpu.VMEM((3, CHUNK, D), jnp.float32),
            pltpu.VMEM((DEG_ROWS,), jnp.float32),
            pltpu.VMEM_SHARED((LOCAL_ROWS, D), jnp.float32),
            pltpu.SemaphoreType.DMA,
            pltpu.SemaphoreType.DMA,
            pltpu.SemaphoreType.DMA,
            pltpu.SemaphoreType.DMA,
        ],
    )(x, packed)


def _tc_body(agg_ref, deg_ref, W_ref, b_ref, out_ref):
    deg = jnp.sum(deg_ref[0, 0], axis=0)[:, None]
    h = agg_ref[0] / jnp.maximum(deg, 1.0)
    out_ref[...] = (
        jnp.dot(h, W_ref[...], preferred_element_type=jnp.float32) + b_ref[...]
    )


@jax.jit
def _tc_finish(agg2, deg_full, W, b2):
    return pl.pallas_call(
        _tc_body,
        grid=(N // TC_BLK,),
        in_specs=[
            pl.BlockSpec((1, TC_BLK, D), lambda i: (i // 5, i % 5, 0)),
            pl.BlockSpec((1, 1, NS, TC_BLK), lambda i: (i // 5, i % 5, 0, 0)),
            pl.BlockSpec((D, D), lambda i: (0, 0)),
            pl.BlockSpec((1, D), lambda i: (0, 0)),
        ],
        out_specs=pl.BlockSpec((TC_BLK, D), lambda i: (i, 0)),
        out_shape=jax.ShapeDtypeStruct((N, D), jnp.float32),
    )(agg2, deg_full, W, b2)


def kernel(x, edge_index, p_map, W, b):
    del p_map
    src = edge_index[0].astype(jnp.int32)
    dst = edge_index[1].astype(jnp.int32)
    pad = E_PAD - E
    packed = jnp.concatenate(
        [src | ((dst + 1) << 14), jnp.zeros((pad,), jnp.int32)])
    agg2, deg2 = _sc_scatter(x, packed)
    deg4 = deg2.reshape(NC, NS, DEG_ROWS)[:, :, :NSPLIT].reshape(
        NC, NS, N // (NC * TC_BLK), TC_BLK).transpose(0, 2, 1, 3)
    return _tc_finish(agg2, deg4, W, b.reshape(1, D))
